# speedup vs baseline: 3.9246x; 3.9246x over previous
import functools

import jax
import jax.numpy as jnp
from jax import lax
from jax.experimental import pallas as pl
from jax.experimental.pallas import tpu as pltpu
from jax.experimental.pallas import tpu_sc as plsc


def _gather_fn(n_total, d_model, n_cores, n_subcores, chunk, n_chunks):
    n_workers = n_cores * n_subcores
    n_per_w = n_total // n_workers

    mesh = plsc.VectorSubcoreMesh(core_axis_name="c", subcore_axis_name="s")

    @functools.partial(
        pl.kernel,
        out_type=jax.ShapeDtypeStruct((n_total, d_model), jnp.float32),
        mesh=mesh,
        scratch_types=[
            pltpu.VMEM((chunk,), jnp.int32),
            pltpu.VMEM((chunk, d_model), jnp.float32),
            pltpu.SemaphoreType.DMA,
        ],
    )
    def run(idx_hbm, table_hbm, out_hbm, idx_v, rows_v, sem):
        wid = lax.axis_index("s") * n_cores + lax.axis_index("c")
        base = wid * n_per_w

        def body(i, carry):
            off = base + i * chunk
            pltpu.sync_copy(idx_hbm.at[pl.ds(off, chunk)], idx_v)
            pltpu.async_copy(table_hbm.at[idx_v], rows_v, sem).wait()
            pltpu.sync_copy(rows_v, out_hbm.at[pl.ds(off, chunk)])
            return carry

        lax.fori_loop(0, n_chunks, body, 0)

    return run


def kernel(positions, pe):
    b, s = positions.shape
    v, d = pe.shape
    n_total = b * s
    idx_flat = positions.reshape(n_total).astype(jnp.int32)

    info = plsc.get_sparse_core_info()
    n_cores, n_subcores = info.num_cores, info.num_subcores
    n_workers = n_cores * n_subcores
    n_per_w = n_total // n_workers
    chunk = 128
    n_chunks = n_per_w // chunk

    out = _gather_fn(n_total, d, n_cores, n_subcores, chunk, n_chunks)(
        idx_flat, pe
    )
    return out.reshape(b, s, d)


# speedup vs baseline: 4.0455x; 1.0308x over previous
import functools

import jax
import jax.numpy as jnp
from jax import lax
from jax.experimental import pallas as pl
from jax.experimental.pallas import tpu as pltpu
from jax.experimental.pallas import tpu_sc as plsc

_NSLOT = 5
_DEPTH = 3


def _gather_fn(n_total, d_model, n_cores, n_subcores, chunk, n_chunks):
    n_workers = n_cores * n_subcores
    n_per_w = n_total // n_workers

    mesh = plsc.VectorSubcoreMesh(core_axis_name="c", subcore_axis_name="s")

    @functools.partial(
        pl.kernel,
        out_type=jax.ShapeDtypeStruct((n_total, d_model), jnp.float32),
        mesh=mesh,
        scratch_types=[
            pltpu.VMEM((n_per_w,), jnp.int32),
            pltpu.VMEM((_NSLOT, chunk, d_model), jnp.float32),
            pltpu.SemaphoreType.DMA((_NSLOT,)),
            pltpu.SemaphoreType.DMA((_NSLOT,)),
        ],
    )
    def run(idx_hbm, table_hbm, out_hbm, idx_v, rows_v, sem_g, sem_s):
        wid = lax.axis_index("s") * n_cores + lax.axis_index("c")
        base = wid * n_per_w

        pltpu.sync_copy(idx_hbm.at[pl.ds(base, n_per_w)], idx_v)

        def gather(i, slot):
            return pltpu.make_async_copy(
                table_hbm.at[idx_v.at[pl.ds(i * chunk, chunk)]],
                rows_v.at[slot],
                sem_g.at[slot],
            )

        def store(i, slot):
            return pltpu.make_async_copy(
                rows_v.at[slot],
                out_hbm.at[pl.ds(base + i * chunk, chunk)],
                sem_s.at[slot],
            )

        for b in range(_DEPTH):
            gather(b, b).start()

        for b in range(_NSLOT):
            gather(b, b).wait()
            store(b, b).start()
            nslot = (b + _DEPTH) % _NSLOT
            if b >= 2:
                store(b - 2, nslot).wait()
            gather(b + _DEPTH, nslot).start()

        def body(g, carry):
            for b in range(_NSLOT):
                i = g * _NSLOT + b
                nslot = (b + _DEPTH) % _NSLOT
                gather(i, b).wait()
                store(i, b).start()
                store(i - 2, nslot).wait()
                gather(i + _DEPTH, nslot).start()
            return carry

        lax.fori_loop(1, n_chunks // _NSLOT - 1, body, 0)

        g_last = n_chunks // _NSLOT - 1
        for b in range(_NSLOT):
            i = g_last * _NSLOT + b
            nslot = (b + _DEPTH) % _NSLOT
            gather(i, b).wait()
            store(i, b).start()
            if i + _DEPTH < n_chunks:
                store(i - 2, nslot).wait()
                gather(i + _DEPTH, nslot).start()

        for b in range(_NSLOT):
            store(g_last * _NSLOT + b, b).wait()

    return run


def kernel(positions, pe):
    b, s = positions.shape
    v, d = pe.shape
    n_total = b * s
    idx_flat = positions.reshape(n_total).astype(jnp.int32)

    info = plsc.get_sparse_core_info()
    n_cores, n_subcores = info.num_cores, info.num_subcores
    n_workers = n_cores * n_subcores
    n_per_w = n_total // n_workers
    chunk = 128
    n_chunks = n_per_w // chunk

    out = _gather_fn(n_total, d, n_cores, n_subcores, chunk, n_chunks)(
        idx_flat, pe
    )
    return out.reshape(b, s, d)
